# Initial kernel scaffold; baseline (speedup 1.0000x reference)
import jax
import jax.numpy as jnp
from jax.experimental import pallas as pl


def kernel(x, num_x, W_num, b_num, a_in, comp1, bases1, root1, bias1, a1, comp2, bases2, root2, bias2, a2, comp3, bases3, root3, bias3, edge_index, edge_type):
    raise NotImplementedError("write your pallas kernel here")


# speedup vs baseline: 3.7430x; 3.7430x over previous
import jax
import jax.numpy as jnp
from jax.experimental import pallas as pl

N = 10000
R = 8


def _prelu(x, a):
    return jnp.where(x >= 0, x, a * x)


def kernel(x, num_x, W_num, b_num, a_in, comp1, bases1, root1, bias1, a1, comp2, bases2, root2, bias2, a2, comp3, bases3, root3, bias3, edge_index, edge_type):
    src = edge_index[0]
    dst = edge_index[1]
    seg = dst * R + edge_type
    nseg = N * R

    cnt = jax.ops.segment_sum(jnp.ones((seg.shape[0],), jnp.float32), seg, num_segments=nseg)
    inv = 1.0 / jnp.maximum(cnt, 1.0)

    h = _prelu(num_x @ W_num + b_num, a_in) + x

    def layer(h, comp, bases, root, bias):
        S = jax.ops.segment_sum(h[src], seg, num_segments=nseg)
        T = S * inv[:, None]
        W = jnp.einsum("rb,bio->rio", comp, bases)
        agg = jnp.einsum("nrd,rdo->no", T.reshape(N, R, -1), W)
        return agg + h @ root + bias

    h = _prelu(layer(h, comp1, bases1, root1, bias1), a1)
    h = _prelu(layer(h, comp2, bases2, root2, bias2), a2)
    h = layer(h, comp3, bases3, root3, bias3)
    return jax.nn.log_softmax(h, axis=1)


# speedup vs baseline: 4.9861x; 1.3321x over previous
import functools

import jax
import jax.numpy as jnp
from jax import lax
from jax.experimental import pallas as pl
from jax.experimental.pallas import tpu as pltpu
from jax.experimental.pallas import tpu_sc as plsc

N = 10000
E = 320000
R = 8
NSEG = N * R
SLABW = 16
NSLAB = 8
NC, NS = 2, 16
EPW = E // NS
BATCH = 125
CHUNKS = EPW // BATCH
ROWS_PT = NSEG // NS
ZROWS = 125


def _sc_segsum(h_t, src_r, seg_r):
    mesh = plsc.VectorSubcoreMesh(core_axis_name="c", subcore_axis_name="s")

    @functools.partial(
        pl.kernel,
        out_type=jax.ShapeDtypeStruct((NSLAB, NSEG, SLABW), jnp.float32),
        mesh=mesh,
        scratch_types=[
            pltpu.VMEM((CHUNKS, BATCH), jnp.int32),
            pltpu.VMEM((CHUNKS, BATCH), jnp.int32),
            pltpu.VMEM((BATCH, SLABW), jnp.float32),
            pltpu.VMEM((ZROWS, SLABW), jnp.float32),
            pltpu.VMEM_SHARED((NSEG, SLABW), jnp.float32),
            pltpu.SemaphoreType.DMA,
        ],
        compiler_params=pltpu.CompilerParams(use_tc_tiling_on_sc=False),
    )
    def k(h_hbm, src_hbm, seg_hbm, out_hbm, idx_v, seg_v, rows_v, zeros_v, accum, sem):
        c = lax.axis_index("c")
        s = lax.axis_index("s")

        pltpu.sync_copy(src_hbm.at[s], idx_v)
        pltpu.sync_copy(seg_hbm.at[s], seg_v)

        def zfill(i, _):
            zeros_v[i] = jnp.zeros((SLABW,), jnp.float32)
            return _
        lax.fori_loop(0, ZROWS, zfill, None)

        for jslab in range(NSLAB // NC):
            slab = c * (NSLAB // NC) + jslab

            def zero_blk(z, _):
                pltpu.sync_copy(
                    zeros_v, accum.at[pl.ds(s * ROWS_PT + z * ZROWS, ZROWS)])
                return _
            lax.fori_loop(0, ROWS_PT // ZROWS, zero_blk, None)
            plsc.subcore_barrier()

            def edge_blk(j, _):
                pltpu.async_copy(
                    h_hbm.at[slab].at[idx_v.at[j]], rows_v, sem).wait()
                pltpu.sync_copy(rows_v, accum.at[seg_v.at[j]], add=True)
                return _
            lax.fori_loop(0, CHUNKS, edge_blk, None)
            plsc.subcore_barrier()

            pltpu.sync_copy(
                accum.at[pl.ds(s * ROWS_PT, ROWS_PT)],
                out_hbm.at[slab].at[pl.ds(s * ROWS_PT, ROWS_PT)])
            plsc.subcore_barrier()

    return k(h_t, src_r, seg_r)


def _prelu(x, a):
    return jnp.where(x >= 0, x, a * x)


def kernel(x, num_x, W_num, b_num, a_in, comp1, bases1, root1, bias1, a1, comp2, bases2, root2, bias2, a2, comp3, bases3, root3, bias3, edge_index, edge_type):
    src = edge_index[0]
    dst = edge_index[1]
    seg = dst * R + edge_type
    src_r = src.reshape(NS, CHUNKS, BATCH)
    seg_r = seg.reshape(NS, CHUNKS, BATCH)

    cnt = jax.ops.segment_sum(jnp.ones((E,), jnp.float32), seg, num_segments=NSEG)
    inv = 1.0 / jnp.maximum(cnt, 1.0)

    h = _prelu(num_x @ W_num + b_num, a_in) + x

    def layer(h, comp, bases, root, bias):
        h_t = h.reshape(N, NSLAB, SLABW).transpose(1, 0, 2)
        S_t = _sc_segsum(h_t, src_r, seg_r)
        T = S_t.transpose(1, 0, 2).reshape(NSEG, -1) * inv[:, None]
        W = jnp.einsum("rb,bio->rio", comp, bases)
        agg = jnp.einsum("nrd,rdo->no", T.reshape(N, R, -1), W)
        return agg + h @ root + bias

    h = _prelu(layer(h, comp1, bases1, root1, bias1), a1)
    h = _prelu(layer(h, comp2, bases2, root2, bias2), a2)
    h = layer(h, comp3, bases3, root3, bias3)
    return jax.nn.log_softmax(h, axis=1)


# speedup vs baseline: 6.9165x; 1.3872x over previous
import functools

import jax
import jax.numpy as jnp
from jax import lax
from jax.experimental import pallas as pl
from jax.experimental.pallas import tpu as pltpu
from jax.experimental.pallas import tpu_sc as plsc

N = 10000
E = 320000
R = 8
NSEG = N * R
SLABW = 16
NSLAB = 8
NC, NS = 2, 16
EPW = E // NS
BATCH = 250
CHUNKS = EPW // BATCH
ROWS_PT = NSEG // NS
ZROWS = 125


def _sc_segsum(h_t, src_r, seg_r):
    mesh = plsc.VectorSubcoreMesh(core_axis_name="c", subcore_axis_name="s")

    @functools.partial(
        pl.kernel,
        out_type=jax.ShapeDtypeStruct((NSLAB, NSEG, SLABW), jnp.float32),
        mesh=mesh,
        scratch_types=[
            pltpu.VMEM((CHUNKS, BATCH), jnp.int32),
            pltpu.VMEM((CHUNKS, BATCH), jnp.int32),
            pltpu.VMEM((BATCH, SLABW), jnp.float32),
            pltpu.VMEM((BATCH, SLABW), jnp.float32),
            pltpu.VMEM((ZROWS, SLABW), jnp.float32),
            pltpu.VMEM_SHARED((NSEG, SLABW), jnp.float32),
            pltpu.SemaphoreType.DMA,
            pltpu.SemaphoreType.DMA,
        ],
        compiler_params=pltpu.CompilerParams(use_tc_tiling_on_sc=False),
    )
    def k(h_hbm, src_hbm, seg_hbm, out_hbm, idx_v, seg_v, rows0_v, rows1_v,
          zeros_v, accum, sem0, sem1):
        c = lax.axis_index("c")
        s = lax.axis_index("s")

        pltpu.sync_copy(src_hbm.at[s], idx_v)
        pltpu.sync_copy(seg_hbm.at[s], seg_v)

        def zfill(i, _):
            zeros_v[i] = jnp.zeros((SLABW,), jnp.float32)
            return _
        lax.fori_loop(0, ZROWS, zfill, None)

        bufs = (rows0_v, rows1_v)
        sems = (sem0, sem1)

        def gather(jc, buf, sem, slab):
            pltpu.async_copy(h_hbm.at[slab].at[idx_v.at[jc]], buf, sem)

        def gwait(buf, sem, slab):
            pltpu.make_async_copy(h_hbm.at[slab].at[idx_v.at[0]], buf, sem).wait()

        for jslab in range(NSLAB // NC):
            slab = c * (NSLAB // NC) + jslab

            def zero_blk(z, _):
                pltpu.sync_copy(
                    zeros_v, accum.at[pl.ds(s * ROWS_PT + z * ZROWS, ZROWS)])
                return _
            lax.fori_loop(0, ROWS_PT // ZROWS, zero_blk, None)
            plsc.subcore_barrier()

            gather(0, bufs[0], sems[0], slab)

            def edge_pair(i, _):
                j0 = 2 * i
                gather(j0 + 1, bufs[1], sems[1], slab)
                gwait(bufs[0], sems[0], slab)
                pltpu.sync_copy(bufs[0], accum.at[seg_v.at[j0]], add=True)
                gather(lax.min(j0 + 2, CHUNKS - 1), bufs[0], sems[0], slab)
                gwait(bufs[1], sems[1], slab)
                pltpu.sync_copy(bufs[1], accum.at[seg_v.at[j0 + 1]], add=True)
                return _
            lax.fori_loop(0, CHUNKS // 2, edge_pair, None)
            gwait(bufs[0], sems[0], slab)
            plsc.subcore_barrier()

            pltpu.sync_copy(
                accum.at[pl.ds(s * ROWS_PT, ROWS_PT)],
                out_hbm.at[slab].at[pl.ds(s * ROWS_PT, ROWS_PT)])
            plsc.subcore_barrier()

    return k(h_t, src_r, seg_r)


def _prelu(x, a):
    return jnp.where(x >= 0, x, a * x)


def kernel(x, num_x, W_num, b_num, a_in, comp1, bases1, root1, bias1, a1, comp2, bases2, root2, bias2, a2, comp3, bases3, root3, bias3, edge_index, edge_type):
    src = edge_index[0]
    dst = edge_index[1]
    seg = dst * R + edge_type
    src_r = src.reshape(NS, CHUNKS, BATCH)
    seg_r = seg.reshape(NS, CHUNKS, BATCH)

    cnt = jax.ops.segment_sum(jnp.ones((E,), jnp.float32), seg, num_segments=NSEG)
    inv = 1.0 / jnp.maximum(cnt, 1.0)

    h = _prelu(num_x @ W_num + b_num, a_in) + x

    def layer(h, comp, bases, root, bias):
        h_t = h.reshape(N, NSLAB, SLABW).transpose(1, 0, 2)
        S_t = _sc_segsum(h_t, src_r, seg_r)
        T = S_t.transpose(1, 0, 2).reshape(NSEG, -1) * inv[:, None]
        W = jnp.einsum("rb,bio->rio", comp, bases)
        agg = jnp.einsum("nrd,rdo->no", T.reshape(N, R, -1), W)
        return agg + h @ root + bias

    h = _prelu(layer(h, comp1, bases1, root1, bias1), a1)
    h = _prelu(layer(h, comp2, bases2, root2, bias2), a2)
    h = layer(h, comp3, bases3, root3, bias3)
    return jax.nn.log_softmax(h, axis=1)


# speedup vs baseline: 14.3575x; 2.0758x over previous
import functools

import jax
import jax.numpy as jnp
from jax import lax
from jax.experimental import pallas as pl
from jax.experimental.pallas import tpu as pltpu
from jax.experimental.pallas import tpu_sc as plsc

N = 10000
E = 320000
R = 8
NSEG = N * R
SLABW = 16
NSLAB = 8
NC, NS = 2, 16
EPW = E // NS
BATCH = 250
CHUNKS = EPW // BATCH
ROWS_PT = NSEG // NS
ZROWS = 125


def _sc_segsum(h8, idx_all, seg_r, with_counts):
    mesh = plsc.VectorSubcoreMesh(core_axis_name="c", subcore_axis_name="s")
    out_type = [jax.ShapeDtypeStruct((NSEG, NSLAB * SLABW), jnp.float32)]
    if with_counts:
        out_type.append(jax.ShapeDtypeStruct((NC, NSEG, SLABW), jnp.float32))

    @functools.partial(
        pl.kernel,
        out_type=tuple(out_type),
        mesh=mesh,
        scratch_types=[
            pltpu.VMEM((CHUNKS, BATCH), jnp.int32),
            pltpu.VMEM((CHUNKS, BATCH), jnp.int32),
            pltpu.VMEM((BATCH, SLABW), jnp.float32),
            pltpu.VMEM((BATCH, SLABW), jnp.float32),
            pltpu.VMEM((ZROWS, SLABW), jnp.float32),
            pltpu.VMEM_SHARED((NSEG, SLABW), jnp.float32),
            pltpu.SemaphoreType.DMA,
            pltpu.SemaphoreType.DMA,
        ],
        compiler_params=pltpu.CompilerParams(use_tc_tiling_on_sc=False),
    )
    def k(h_hbm, idx_hbm, seg_hbm, *refs):
        if with_counts:
            (s_hbm, cnt_hbm, idx_v, seg_v, rows0_v, rows1_v, zeros_v, accum,
             sem0, sem1) = refs
        else:
            (s_hbm, idx_v, seg_v, rows0_v, rows1_v, zeros_v, accum,
             sem0, sem1) = refs
            cnt_hbm = None
        c = lax.axis_index("c")
        s = lax.axis_index("s")

        pltpu.sync_copy(seg_hbm.at[s], seg_v)

        def zfill(i, _):
            zeros_v[i] = jnp.zeros((SLABW,), jnp.float32)
            return _
        lax.fori_loop(0, ZROWS, zfill, None)

        def zero_accum():
            def zero_blk(z, _):
                pltpu.sync_copy(
                    zeros_v, accum.at[pl.ds(s * ROWS_PT + z * ZROWS, ZROWS)])
                return _
            lax.fori_loop(0, ROWS_PT // ZROWS, zero_blk, None)
            plsc.subcore_barrier()

        if with_counts:
            def ofill(i, _):
                rows0_v[i] = jnp.ones((SLABW,), jnp.float32)
                return _
            lax.fori_loop(0, BATCH, ofill, None)
            zero_accum()

            def cnt_blk(j, _):
                pltpu.sync_copy(
                    rows0_v, accum.at[seg_v.at[c * (CHUNKS // NC) + j]],
                    add=True)
                return _
            lax.fori_loop(0, CHUNKS // NC, cnt_blk, None)
            plsc.subcore_barrier()
            pltpu.sync_copy(
                accum.at[pl.ds(s * ROWS_PT, ROWS_PT)],
                cnt_hbm.at[c].at[pl.ds(s * ROWS_PT, ROWS_PT)])
            plsc.subcore_barrier()

        def gather(jc, buf, sem, slab):
            pltpu.async_copy(h_hbm.at[idx_v.at[jc]], buf, sem)

        def gwait(buf, sem):
            pltpu.make_async_copy(h_hbm.at[idx_v.at[0]], buf, sem).wait()

        for jslab in range(NSLAB // NC):
            slab = c * (NSLAB // NC) + jslab
            pltpu.sync_copy(idx_hbm.at[slab].at[s], idx_v)
            zero_accum()

            gather(0, rows0_v, sem0, slab)

            def edge_pair(i, _):
                j0 = 2 * i
                gather(j0 + 1, rows1_v, sem1, slab)
                gwait(rows0_v, sem0)
                pltpu.sync_copy(rows0_v, accum.at[seg_v.at[j0]], add=True)
                gather(lax.min(j0 + 2, CHUNKS - 1), rows0_v, sem0, slab)
                gwait(rows1_v, sem1)
                pltpu.sync_copy(rows1_v, accum.at[seg_v.at[j0 + 1]], add=True)
                return _
            lax.fori_loop(0, CHUNKS // 2, edge_pair, None)
            gwait(rows0_v, sem0)
            plsc.subcore_barrier()

            pltpu.sync_copy(
                accum.at[pl.ds(s * ROWS_PT, ROWS_PT)],
                s_hbm.at[pl.ds(s * ROWS_PT, ROWS_PT),
                         pl.ds(SLABW * slab, SLABW)])
            plsc.subcore_barrier()

    return k(h8, idx_all, seg_r)


def _prelu(x, a):
    return jnp.where(x >= 0, x, a * x)


def kernel(x, num_x, W_num, b_num, a_in, comp1, bases1, root1, bias1, a1, comp2, bases2, root2, bias2, a2, comp3, bases3, root3, bias3, edge_index, edge_type):
    src = edge_index[0]
    dst = edge_index[1]
    seg = dst * R + edge_type
    idx_all = (src * NSLAB)[None, :] + jnp.arange(NSLAB, dtype=jnp.int32)[:, None]
    idx_all = idx_all.reshape(NSLAB, NS, CHUNKS, BATCH)
    seg_r = seg.reshape(NS, CHUNKS, BATCH)

    h = _prelu(num_x @ W_num + b_num, a_in) + x

    def layer(h, comp, bases, root, bias, with_counts, inv):
        res = _sc_segsum(h.reshape(N * NSLAB, SLABW), idx_all, seg_r,
                         with_counts)
        if with_counts:
            S, cnt16 = res
            cnt = cnt16[0, :, 0] + cnt16[1, :, 0]
            inv = 1.0 / jnp.maximum(cnt, 1.0)
        else:
            (S,) = res
        T = S * inv[:, None]
        W = jnp.einsum("rb,bio->rio", comp, bases)
        agg = jnp.einsum("nrd,rdo->no", T.reshape(N, R, -1), W)
        return agg + h @ root + bias, inv

    h1, inv = layer(h, comp1, bases1, root1, bias1, True, None)
    h = _prelu(h1, a1)
    h2, _ = layer(h, comp2, bases2, root2, bias2, False, inv)
    h = _prelu(h2, a2)
    h3, _ = layer(h, comp3, bases3, root3, bias3, False, inv)
    return jax.nn.log_softmax(h3, axis=1)
